# baseline (speedup 1.0000x reference)
import functools

import jax
import jax.numpy as jnp
from jax import lax
from jax.experimental import pallas as pl
from jax.experimental.pallas import tpu as pltpu
from jax.experimental.pallas import tpu_sc as plsc

VOCAB = 100000
DIM = 16
BATCH = 1024


_INFO = plsc.get_sparse_core_info()
_NC, _NS = _INFO.num_cores, _INFO.num_subcores
_NW = _NC * _NS
_B2 = 2 * BATCH
_BPW = _B2 // _NW


@functools.partial(
    pl.kernel,
    mesh=plsc.VectorSubcoreMesh(core_axis_name="c", subcore_axis_name="s"),
    out_type=jax.ShapeDtypeStruct((_B2, DIM), jnp.float32),
    scratch_types=[
        pltpu.VMEM((_BPW,), jnp.int32),
        pltpu.VMEM((_BPW, DIM), jnp.float32),
        pltpu.SemaphoreType.DMA,
    ],
    compiler_params=pltpu.CompilerParams(use_tc_tiling_on_sc=False),
)
def _sc_gather(table_hbm, idx_hbm, out_hbm, idx_v, rows_v, sem):
    wid = lax.axis_index("s") * _NC + lax.axis_index("c")
    base = wid * _BPW
    pltpu.sync_copy(idx_hbm.at[pl.ds(base, _BPW)], idx_v)
    pltpu.async_copy(table_hbm.at[idx_v], rows_v, sem).wait()
    pltpu.sync_copy(rows_v, out_hbm.at[pl.ds(base, _BPW)])


_VB = 512


def _mm_body(xi_ref, xj_ref, w1_ref, b1_ref, w2_ref, b2_ref, o1_ref, o2_ref):
    dn = (((1,), (1,)), ((), ()))
    o1_ref[...] = (
        lax.dot_general(xi_ref[...], w1_ref[...], dn,
                        preferred_element_type=jnp.float32)
        + b1_ref[...]
    )
    o2_ref[...] = (
        lax.dot_general(xj_ref[...], w2_ref[...], dn,
                        preferred_element_type=jnp.float32)
        + b2_ref[...]
    )


def _tc_matmuls(xi, xj, W1, b1, W2, b2):
    grid = (pl.cdiv(VOCAB, _VB),)
    return pl.pallas_call(
        _mm_body,
        grid=grid,
        in_specs=[
            pl.BlockSpec((BATCH, DIM), lambda v: (0, 0)),
            pl.BlockSpec((BATCH, DIM), lambda v: (0, 0)),
            pl.BlockSpec((_VB, DIM), lambda v: (v, 0)),
            pl.BlockSpec((1, _VB), lambda v: (0, v)),
            pl.BlockSpec((_VB, DIM), lambda v: (v, 0)),
            pl.BlockSpec((1, _VB), lambda v: (0, v)),
        ],
        out_specs=[
            pl.BlockSpec((BATCH, _VB), lambda v: (0, v)),
            pl.BlockSpec((BATCH, _VB), lambda v: (0, v)),
        ],
        out_shape=[
            jax.ShapeDtypeStruct((BATCH, VOCAB), jnp.float32),
            jax.ShapeDtypeStruct((BATCH, VOCAB), jnp.float32),
        ],
        compiler_params=pltpu.CompilerParams(
            dimension_semantics=("parallel",),
        ),
    )(xi, xj, W1, b1.reshape(1, VOCAB), W2, b2.reshape(1, VOCAB))


def kernel(i_indices, j_indices, emb, W1, b1, W2, b2):
    idx = jnp.concatenate(
        [i_indices.astype(jnp.int32), j_indices.astype(jnp.int32)]
    )
    rows = _sc_gather(emb, idx)
    xi = rows[:BATCH]
    xj = rows[BATCH:]
    return _tc_matmuls(xi, xj, W1, b1, W2, b2)


# speedup vs baseline: 1.1299x; 1.1299x over previous
import functools

import jax
import jax.numpy as jnp
from jax import lax
from jax.experimental import pallas as pl
from jax.experimental.pallas import tpu as pltpu
from jax.experimental.pallas import tpu_sc as plsc

VOCAB = 100000
DIM = 16
BATCH = 1024


_INFO = plsc.get_sparse_core_info()
_NC, _NS = _INFO.num_cores, _INFO.num_subcores
_NW = _NC * _NS
_B2 = 2 * BATCH
_BPW = _B2 // _NW


@functools.partial(
    pl.kernel,
    mesh=plsc.VectorSubcoreMesh(core_axis_name="c", subcore_axis_name="s"),
    out_type=jax.ShapeDtypeStruct((_B2, DIM), jnp.float32),
    scratch_types=[
        pltpu.VMEM((_BPW,), jnp.int32),
        pltpu.VMEM((_BPW, DIM), jnp.float32),
        pltpu.SemaphoreType.DMA,
    ],
    compiler_params=pltpu.CompilerParams(use_tc_tiling_on_sc=False),
)
def _sc_gather(table_hbm, idx_hbm, out_hbm, idx_v, rows_v, sem):
    wid = lax.axis_index("s") * _NC + lax.axis_index("c")
    base = wid * _BPW
    pltpu.sync_copy(idx_hbm.at[pl.ds(base, _BPW)], idx_v)
    pltpu.async_copy(table_hbm.at[idx_v], rows_v, sem).wait()
    pltpu.sync_copy(rows_v, out_hbm.at[pl.ds(base, _BPW)])


_RB = 16


def _mm_body(xi_ref, xj_ref, w1t_ref, b1_ref, w2t_ref, b2_ref, o1_ref, o2_ref):
    dn = (((1,), (0,)), ((), ()))
    o1_ref[...] = (
        lax.dot_general(xi_ref[...], w1t_ref[...], dn,
                        preferred_element_type=jnp.float32)
        + b1_ref[...]
    )
    o2_ref[...] = (
        lax.dot_general(xj_ref[...], w2t_ref[...], dn,
                        preferred_element_type=jnp.float32)
        + b2_ref[...]
    )


def _tc_matmuls(xi, xj, W1, b1, W2, b2):
    grid = (BATCH // _RB,)
    return pl.pallas_call(
        _mm_body,
        grid=grid,
        in_specs=[
            pl.BlockSpec((_RB, DIM), lambda v: (v, 0)),
            pl.BlockSpec((_RB, DIM), lambda v: (v, 0)),
            pl.BlockSpec((DIM, VOCAB), lambda v: (0, 0)),
            pl.BlockSpec((1, VOCAB), lambda v: (0, 0)),
            pl.BlockSpec((DIM, VOCAB), lambda v: (0, 0)),
            pl.BlockSpec((1, VOCAB), lambda v: (0, 0)),
        ],
        out_specs=[
            pl.BlockSpec((_RB, VOCAB), lambda v: (v, 0)),
            pl.BlockSpec((_RB, VOCAB), lambda v: (v, 0)),
        ],
        out_shape=[
            jax.ShapeDtypeStruct((BATCH, VOCAB), jnp.float32),
            jax.ShapeDtypeStruct((BATCH, VOCAB), jnp.float32),
        ],
        compiler_params=pltpu.CompilerParams(
            dimension_semantics=("parallel",),
        ),
    )(xi, xj, W1.T, b1.reshape(1, VOCAB), W2.T, b2.reshape(1, VOCAB))


def kernel(i_indices, j_indices, emb, W1, b1, W2, b2):
    idx = jnp.concatenate(
        [i_indices.astype(jnp.int32), j_indices.astype(jnp.int32)]
    )
    rows = _sc_gather(emb, idx)
    xi = rows[:BATCH]
    xj = rows[BATCH:]
    return _tc_matmuls(xi, xj, W1, b1, W2, b2)


# speedup vs baseline: 1.1336x; 1.0033x over previous
import functools

import jax
import jax.numpy as jnp
from jax import lax
from jax.experimental import pallas as pl
from jax.experimental.pallas import tpu as pltpu
from jax.experimental.pallas import tpu_sc as plsc

VOCAB = 100000
DIM = 16
BATCH = 1024


_INFO = plsc.get_sparse_core_info()
_NC, _NS = _INFO.num_cores, _INFO.num_subcores
_NW = _NC * _NS
_B2 = 2 * BATCH
_BPW = _B2 // _NW


@functools.partial(
    pl.kernel,
    mesh=plsc.VectorSubcoreMesh(core_axis_name="c", subcore_axis_name="s"),
    out_type=jax.ShapeDtypeStruct((_B2, DIM), jnp.float32),
    scratch_types=[
        pltpu.VMEM((_BPW,), jnp.int32),
        pltpu.VMEM((_BPW, DIM), jnp.float32),
        pltpu.SemaphoreType.DMA,
    ],
    compiler_params=pltpu.CompilerParams(use_tc_tiling_on_sc=False),
)
def _sc_gather(table_hbm, idx_hbm, out_hbm, idx_v, rows_v, sem):
    wid = lax.axis_index("s") * _NC + lax.axis_index("c")
    base = wid * _BPW
    pltpu.sync_copy(idx_hbm.at[pl.ds(base, _BPW)], idx_v)
    pltpu.async_copy(table_hbm.at[idx_v], rows_v, sem).wait()
    pltpu.sync_copy(rows_v, out_hbm.at[pl.ds(base, _BPW)])


_RB = 16
_NSTEPS = BATCH // _RB
_NBUF = 3


def _mm_body(xi_ref, xj_ref, w1t_ref, b1_ref, w2t_ref, b2_ref,
             o1_hbm, o2_hbm, o1_buf, o2_buf, sem1, sem2):
    i = pl.program_id(0)
    nb = lax.rem(i, _NBUF)
    dn = (((1,), (0,)), ((), ()))

    @pl.when(i >= _NBUF)
    def _drain_oldest():
        j = i - _NBUF
        pltpu.make_async_copy(
            o1_buf.at[nb], o1_hbm.at[pl.ds(j * _RB, _RB)], sem1.at[nb]).wait()
        pltpu.make_async_copy(
            o2_buf.at[nb], o2_hbm.at[pl.ds(j * _RB, _RB)], sem2.at[nb]).wait()

    o1_buf[nb] = (
        lax.dot_general(xi_ref[...], w1t_ref[...], dn,
                        preferred_element_type=jnp.float32)
        + b1_ref[...]
    )
    o2_buf[nb] = (
        lax.dot_general(xj_ref[...], w2t_ref[...], dn,
                        preferred_element_type=jnp.float32)
        + b2_ref[...]
    )
    pltpu.make_async_copy(
        o1_buf.at[nb], o1_hbm.at[pl.ds(i * _RB, _RB)], sem1.at[nb]).start()
    pltpu.make_async_copy(
        o2_buf.at[nb], o2_hbm.at[pl.ds(i * _RB, _RB)], sem2.at[nb]).start()

    @pl.when(i == _NSTEPS - 1)
    def _drain_all():
        for k in range(_NBUF):
            j = _NSTEPS - 1 - k
            b = lax.rem(jnp.int32(j), _NBUF)
            pltpu.make_async_copy(
                o1_buf.at[b], o1_hbm.at[pl.ds(j * _RB, _RB)], sem1.at[b]).wait()
            pltpu.make_async_copy(
                o2_buf.at[b], o2_hbm.at[pl.ds(j * _RB, _RB)], sem2.at[b]).wait()


def _tc_matmuls(xi, xj, W1, b1, W2, b2):
    grid = (_NSTEPS,)
    return pl.pallas_call(
        _mm_body,
        grid=grid,
        in_specs=[
            pl.BlockSpec((_RB, DIM), lambda v: (v, 0)),
            pl.BlockSpec((_RB, DIM), lambda v: (v, 0)),
            pl.BlockSpec((DIM, VOCAB), lambda v: (0, 0)),
            pl.BlockSpec((1, VOCAB), lambda v: (0, 0)),
            pl.BlockSpec((DIM, VOCAB), lambda v: (0, 0)),
            pl.BlockSpec((1, VOCAB), lambda v: (0, 0)),
        ],
        out_specs=[
            pl.BlockSpec(memory_space=pl.ANY),
            pl.BlockSpec(memory_space=pl.ANY),
        ],
        out_shape=[
            jax.ShapeDtypeStruct((BATCH, VOCAB), jnp.float32),
            jax.ShapeDtypeStruct((BATCH, VOCAB), jnp.float32),
        ],
        scratch_shapes=[
            pltpu.VMEM((_NBUF, _RB, VOCAB), jnp.float32),
            pltpu.VMEM((_NBUF, _RB, VOCAB), jnp.float32),
            pltpu.SemaphoreType.DMA((_NBUF,)),
            pltpu.SemaphoreType.DMA((_NBUF,)),
        ],
        compiler_params=pltpu.CompilerParams(
            dimension_semantics=("arbitrary",),
        ),
    )(xi, xj, W1.T, b1.reshape(1, VOCAB), W2.T, b2.reshape(1, VOCAB))


def kernel(i_indices, j_indices, emb, W1, b1, W2, b2):
    idx = jnp.concatenate(
        [i_indices.astype(jnp.int32), j_indices.astype(jnp.int32)]
    )
    rows = _sc_gather(emb, idx)
    xi = rows[:BATCH]
    xj = rows[BATCH:]
    return _tc_matmuls(xi, xj, W1, b1, W2, b2)
